# Initial kernel scaffold; baseline (speedup 1.0000x reference)
import jax
import jax.numpy as jnp
from jax.experimental import pallas as pl


def kernel(x, rest_ast_edge_index, edges_remove_0, edges_remove_1, W, b):
    raise NotImplementedError("write your pallas kernel here")


# speedup vs baseline: 3.1204x; 3.1204x over previous
import functools
import math

import jax
import jax.numpy as jnp
import numpy as np
from jax import lax
from jax.experimental import pallas as pl
from jax.experimental.pallas import tpu as pltpu
from jax.experimental.pallas import tpu_sc as plsc

N = 10000
D = 256
NC = 2
NS = 16
NW = NC * NS
CH = 512
NPAD = 10240
BR = 1024
FPT = 4
PASSES = D // (NW * FPT)

_SC_PARAMS = pltpu.CompilerParams(needs_layout_passes=False)


def _pad_edges(e):
    ecount = e.shape[1]
    ep = ((ecount + NW * CH - 1) // (NW * CH)) * (NW * CH)
    pad = ep - ecount
    src = jnp.concatenate([e[0], jnp.zeros((pad,), jnp.int32)])
    dst = jnp.concatenate([e[1], jnp.full((pad,), N, jnp.int32)])
    return src, dst, ep


def _deg_body(eps, dsts, outs, dstb, hist):
    c = lax.axis_index("c")
    s = lax.axis_index("s")
    w = c * NS + s
    zeros16 = jnp.zeros((16,), jnp.float32)
    ones16 = jnp.full((16,), 1.0, jnp.float32)
    for ep, dst_hbm, out_hbm in zip(eps, dsts, outs):
        def zero(k, carry):
            hist[pl.ds(k * 16, 16)] = zeros16
            return carry
        lax.fori_loop(0, NPAD // 16, zero, 0)
        pt = ep // NW

        def chunk(k, carry):
            off = w * pt + k * CH
            pltpu.sync_copy(dst_hbm.at[pl.ds(off, CH)], dstb)
            for g in range(CH // 16):
                dv = dstb[pl.ds(g * 16, 16)]
                plsc.addupdate_scatter(hist, [dv], ones16)
            return carry

        lax.fori_loop(0, pt // CH, chunk, 0)
        pltpu.sync_copy(hist, out_hbm.at[w])


def _deg_call(dst0, dst1, dst2, ep0, ep1, ep2):
    mesh = plsc.VectorSubcoreMesh(core_axis_name="c", subcore_axis_name="s",
                                  num_cores=NC, num_subcores=NS)

    def body(dst0, dst1, dst2, out0, out1, out2, dstb, hist):
        _deg_body((ep0, ep1, ep2), (dst0, dst1, dst2), (out0, out1, out2),
                  dstb, hist)

    k = pl.kernel(
        body,
        compiler_params=_SC_PARAMS,
        out_type=[jax.ShapeDtypeStruct((NW, NPAD), jnp.float32)] * 3,
        mesh=mesh,
        scratch_types=[
            pltpu.VMEM((CH,), jnp.int32),
            pltpu.VMEM((NPAD,), jnp.float32),
        ],
    )
    return k(dst0, dst1, dst2)


def _scatter_body(ep, gt_hbm, src_hbm, dst_hbm, out_hbm,
                  srcb, dstb, gts, accs):
    c = lax.axis_index("c")
    s = lax.axis_index("s")
    w = c * NS + s
    nchunks = ep // CH
    for p in range(PASSES):
        fbase = p * (D // PASSES) + w * FPT
        for j in range(FPT):
            pltpu.sync_copy(gt_hbm.at[fbase + j], gts[j])
            pltpu.sync_copy(gt_hbm.at[fbase + j], accs[j])

        def chunk(k, carry):
            kk = k + w * (nchunks // NW)
            kk = jnp.where(kk >= nchunks, kk - nchunks, kk)
            off = kk * CH
            pltpu.sync_copy(src_hbm.at[pl.ds(off, CH)], srcb)
            pltpu.sync_copy(dst_hbm.at[pl.ds(off, CH)], dstb)
            for g in range(CH // 16):
                sl = pl.ds(g * 16, 16)
                sv = srcb[sl]
                dv = dstb[sl]
                for j in range(FPT):
                    vals = plsc.load_gather(gts[j], [sv])
                    plsc.addupdate_scatter(accs[j], [dv], vals)
            return carry

        lax.fori_loop(0, nchunks, chunk, 0)
        for j in range(FPT):
            pltpu.sync_copy(accs[j], out_hbm.at[fbase + j])


def _scatter_call(gt, src, dst, ep):
    mesh = plsc.VectorSubcoreMesh(core_axis_name="c", subcore_axis_name="s",
                                  num_cores=NC, num_subcores=NS)

    def body(gt_h, src_h, dst_h, out_h, srcb, dstb,
             gt0, gt1, gt2, gt3, ac0, ac1, ac2, ac3):
        _scatter_body(ep, gt_h, src_h, dst_h, out_h, srcb, dstb,
                      (gt0, gt1, gt2, gt3), (ac0, ac1, ac2, ac3))

    k = pl.kernel(
        body,
        compiler_params=_SC_PARAMS,
        out_type=jax.ShapeDtypeStruct((D, NPAD), jnp.float32),
        mesh=mesh,
        scratch_types=[pltpu.VMEM((CH,), jnp.int32)] * 2
        + [pltpu.VMEM((NPAD,), jnp.float32)] * (2 * FPT),
    )
    return k(gt, src, dst)


def _dinv(deg_blk):
    deg = jnp.sum(deg_blk, axis=0) + 1.0
    return lax.rsqrt(jnp.maximum(deg, 1e-12))[None, :]


def _pre_body(x_ref, w_ref, deg_ref, g_ref):
    h = jnp.dot(w_ref[...], x_ref[...], preferred_element_type=jnp.float32)
    g_ref[...] = h * _dinv(deg_ref[...])


def _pre_call(xt, wt, deg):
    grid = (NPAD // BR,)
    blk = pl.BlockSpec((D, BR), lambda i: (0, i))
    return pl.pallas_call(
        _pre_body,
        grid=grid,
        in_specs=[
            blk,
            pl.BlockSpec((D, D), lambda i: (0, 0)),
            pl.BlockSpec((NW, BR), lambda i: (0, i)),
        ],
        out_specs=blk,
        out_shape=jax.ShapeDtypeStruct((D, NPAD), jnp.float32),
    )(xt, wt, deg)


def _ln_gelu(t):
    mu = jnp.mean(t, axis=0, keepdims=True)
    dlt = t - mu
    var = jnp.mean(dlt * dlt, axis=0, keepdims=True)
    y = dlt * lax.rsqrt(var + 1e-5)
    return 0.5 * y * (1.0 + lax.erf(y * np.float32(1.0 / math.sqrt(2.0))))


def _post_body(acc_ref, deg_ref, b_ref, r1_ref, out_ref):
    t = acc_ref[...] * _dinv(deg_ref[...]) + b_ref[:, 0:1]
    out_ref[...] = _ln_gelu(t) + r1_ref[...]


def _post2_body(acc_ref, deg_ref, b_ref, r1_ref, r2_ref, out_ref):
    t = acc_ref[...] * _dinv(deg_ref[...]) + b_ref[:, 0:1]
    out_ref[...] = _ln_gelu(t) + r1_ref[...] + r2_ref[...]


def _final_body(acc_ref, deg_ref, b_ref, out_ref):
    t = acc_ref[...] * _dinv(deg_ref[...]) + b_ref[:, 0:1]
    out_ref[...] = jnp.maximum(t, 0.0)


def _post_call(body, acc, deg, bt, *residuals):
    grid = (NPAD // BR,)
    blk = pl.BlockSpec((D, BR), lambda i: (0, i))
    in_specs = [
        blk,
        pl.BlockSpec((NW, BR), lambda i: (0, i)),
        pl.BlockSpec((D, 8), lambda i: (0, 0)),
    ] + [blk] * len(residuals)
    return pl.pallas_call(
        body,
        grid=grid,
        in_specs=in_specs,
        out_specs=blk,
        out_shape=jax.ShapeDtypeStruct((D, NPAD), jnp.float32),
    )(acc, deg, bt, *residuals)


def kernel(x, rest_ast_edge_index, edges_remove_0, edges_remove_1, W, b):
    xt = jnp.pad(x.T, ((0, 0), (0, NPAD - N)))
    wt = W.T
    bt = jnp.broadcast_to(b[:, None], (D, 8))
    src0, dst0, ep0 = _pad_edges(edges_remove_0)
    src1, dst1, ep1 = _pad_edges(edges_remove_1)
    src2, dst2, ep2 = _pad_edges(rest_ast_edge_index)

    deg0, deg1, deg2 = _deg_call(dst0, dst1, dst2, ep0, ep1, ep2)

    g0 = _pre_call(xt, wt, deg0)
    acc0 = _scatter_call(g0, src0, dst0, ep0)
    x1 = _post_call(_post_body, acc0, deg0, bt, xt)

    g1 = _pre_call(x1, wt, deg1)
    acc1 = _scatter_call(g1, src1, dst1, ep1)
    x2 = _post_call(_post2_body, acc1, deg1, bt, x1, xt)

    g2 = _pre_call(x2, wt, deg2)
    acc2 = _scatter_call(g2, src2, dst2, ep2)
    out = _post_call(_final_body, acc2, deg2, bt)
    return out[:, :N].T


# speedup vs baseline: 6.2377x; 1.9990x over previous
import functools
import math

import jax
import jax.numpy as jnp
import numpy as np
from jax import lax
from jax.experimental import pallas as pl
from jax.experimental.pallas import tpu as pltpu
from jax.experimental.pallas import tpu_sc as plsc

N = 10000
D = 256
NC = 2
NS = 16
NW = NC * NS
CH = 512
NPAD = 10240
BR = 1024
FPT = 4
PASSES = D // (NW * FPT)

_SC_PARAMS = pltpu.CompilerParams(needs_layout_passes=False)


def _pad_edges(e):
    ecount = e.shape[1]
    ep = ((ecount + NW * CH - 1) // (NW * CH)) * (NW * CH)
    pad = ep - ecount
    src = jnp.concatenate([e[0], jnp.zeros((pad,), jnp.int32)])
    dst = jnp.concatenate([e[1], jnp.full((pad,), N, jnp.int32)])
    return src, dst, ep


def _deg_hist(w, ep, dst_hbm, out_ref, dstb, hist):
    zeros16 = jnp.zeros((16,), jnp.float32)
    ones16 = jnp.full((16,), 1.0, jnp.float32)

    def zero(k, carry):
        hist[pl.ds(k * 16, 16)] = zeros16
        return carry

    lax.fori_loop(0, NPAD // 16, zero, 0)
    pt = ep // NW

    def chunk(k, carry):
        off = w * pt + k * CH
        pltpu.sync_copy(dst_hbm.at[pl.ds(off, CH)], dstb)
        for g in range(CH // 16):
            dv = dstb[pl.ds(g * 16, 16)]
            plsc.addupdate_scatter(hist, [dv], ones16)
        return carry

    lax.fori_loop(0, pt // CH, chunk, 0)
    pltpu.sync_copy(hist, out_ref)


def _deg_call(dst0, dst1, dst2, ep0, ep1, ep2):
    mesh = plsc.VectorSubcoreMesh(core_axis_name="c", subcore_axis_name="s",
                                  num_cores=NC, num_subcores=NS)
    eps = (ep0, ep1, ep2)

    def body(d0, d1, d2, out0, dstb, hist):
        c = lax.axis_index("c")
        s = lax.axis_index("s")
        w = c * NS + s
        for e, dh in enumerate((d0, d1, d2)):
            _deg_hist(w, eps[e], dh, out0.at[e, w], dstb, hist)

    k = pl.kernel(
        body,
        compiler_params=_SC_PARAMS,
        out_type=jax.ShapeDtypeStruct((3, NW, NPAD), jnp.float32),
        mesh=mesh,
        scratch_types=[
            pltpu.VMEM((CH,), jnp.int32),
            pltpu.VMEM((NPAD,), jnp.float32),
        ],
    )
    degs = k(dst0, dst1, dst2)
    return degs[0], degs[1], degs[2]


SCH = 1024


def _scatter_body(ep, gt_hbm, src_hbm, dst_hbm, out_hbm,
                  srcb, dstb, gts, accs, sems):
    c = lax.axis_index("c")
    s = lax.axis_index("s")
    w = c * NS + s
    nchunks = ep // SCH
    shift = (w * nchunks) // NW

    def off(kk):
        kk = kk + shift
        kk = jnp.where(kk >= nchunks, kk - nchunks, kk)
        return kk * SCH

    def start(b, k):
        pltpu.async_copy(src_hbm.at[pl.ds(off(k), SCH)], srcb.at[b], sems[b])
        pltpu.async_copy(dst_hbm.at[pl.ds(off(k), SCH)], dstb.at[b], sems[b])

    def drain(b):
        pltpu.make_async_copy(src_hbm.at[pl.ds(0, SCH)], srcb.at[b], sems[b]).wait()
        pltpu.make_async_copy(dst_hbm.at[pl.ds(0, SCH)], dstb.at[b], sems[b]).wait()

    for p in range(PASSES):
        fbase = p * (D // PASSES) + w * FPT
        for j in range(FPT):
            pltpu.sync_copy(gt_hbm.at[fbase + j], gts[j])
            pltpu.sync_copy(gt_hbm.at[fbase + j], accs[j])
        for b in range(2):
            start(b, b)

        def pair(k2, carry):
            k = k2 * 2
            for b in range(2):
                drain(b)
                for g in range(SCH // 16):
                    sl = pl.ds(g * 16, 16)
                    sv = srcb[b, sl]
                    dv = dstb[b, sl]
                    for j in range(FPT):
                        vals = plsc.load_gather(gts[j], [sv])
                        plsc.addupdate_scatter(accs[j], [dv], vals)
                nxt = k + b + 2

                @pl.when(nxt < nchunks)
                def _():
                    start(b, nxt)
            return carry

        lax.fori_loop(0, nchunks // 2, pair, 0)
        for j in range(FPT):
            pltpu.sync_copy(accs[j], out_hbm.at[fbase + j])


def _scatter_call(gt, src, dst, ep):
    mesh = plsc.VectorSubcoreMesh(core_axis_name="c", subcore_axis_name="s",
                                  num_cores=NC, num_subcores=NS)

    def body(gt_h, src_h, dst_h, out_h, srcb, dstb, sem0, sem1,
             gt0, gt1, gt2, gt3, ac0, ac1, ac2, ac3):
        _scatter_body(ep, gt_h, src_h, dst_h, out_h, srcb, dstb,
                      (gt0, gt1, gt2, gt3), (ac0, ac1, ac2, ac3),
                      (sem0, sem1))

    k = pl.kernel(
        body,
        compiler_params=_SC_PARAMS,
        out_type=jax.ShapeDtypeStruct((D, NPAD), jnp.float32),
        mesh=mesh,
        scratch_types=[pltpu.VMEM((2, SCH), jnp.int32)] * 2
        + [pltpu.SemaphoreType.DMA] * 2
        + [pltpu.VMEM((NPAD,), jnp.float32)] * (2 * FPT),
    )
    return k(gt, src, dst)


def _dinv(deg_blk):
    deg = jnp.sum(deg_blk, axis=0) + 1.0
    return lax.rsqrt(jnp.maximum(deg, 1e-12))[None, :]


def _pre_body(x_ref, w_ref, deg_ref, g_ref):
    h = jnp.dot(w_ref[...], x_ref[...], preferred_element_type=jnp.float32)
    g_ref[...] = h * _dinv(deg_ref[...])


def _pre_call(xt, wt, deg):
    grid = (NPAD // BR,)
    blk = pl.BlockSpec((D, BR), lambda i: (0, i))
    return pl.pallas_call(
        _pre_body,
        grid=grid,
        in_specs=[
            blk,
            pl.BlockSpec((D, D), lambda i: (0, 0)),
            pl.BlockSpec((NW, BR), lambda i: (0, i)),
        ],
        out_specs=blk,
        out_shape=jax.ShapeDtypeStruct((D, NPAD), jnp.float32),
    )(xt, wt, deg)


def _ln_gelu(t):
    mu = jnp.mean(t, axis=0, keepdims=True)
    dlt = t - mu
    var = jnp.mean(dlt * dlt, axis=0, keepdims=True)
    y = dlt * lax.rsqrt(var + 1e-5)
    return 0.5 * y * (1.0 + lax.erf(y * np.float32(1.0 / math.sqrt(2.0))))


def _postpre_body(acc_ref, deg_ref, degn_ref, b_ref, w_ref, r1_ref,
                  x_ref, g_ref):
    t = acc_ref[...] * _dinv(deg_ref[...]) + b_ref[:, 0:1]
    xn = _ln_gelu(t) + r1_ref[...]
    x_ref[...] = xn
    h = jnp.dot(w_ref[...], xn, preferred_element_type=jnp.float32)
    g_ref[...] = h * _dinv(degn_ref[...])


def _postpre2_body(acc_ref, deg_ref, degn_ref, b_ref, w_ref, r1_ref, r2_ref,
                   x_ref, g_ref):
    t = acc_ref[...] * _dinv(deg_ref[...]) + b_ref[:, 0:1]
    xn = _ln_gelu(t) + r1_ref[...] + r2_ref[...]
    x_ref[...] = xn
    h = jnp.dot(w_ref[...], xn, preferred_element_type=jnp.float32)
    g_ref[...] = h * _dinv(degn_ref[...])


def _final_body(acc_ref, deg_ref, b_ref, out_ref):
    t = acc_ref[...] * _dinv(deg_ref[...]) + b_ref[:, 0:1]
    out_ref[...] = jnp.maximum(t, 0.0)


def _postpre_call(body, acc, deg, degn, bt, wt, *residuals):
    grid = (NPAD // BR,)
    blk = pl.BlockSpec((D, BR), lambda i: (0, i))
    dblk = pl.BlockSpec((NW, BR), lambda i: (0, i))
    in_specs = [
        blk,
        dblk,
        dblk,
        pl.BlockSpec((D, 8), lambda i: (0, 0)),
        pl.BlockSpec((D, D), lambda i: (0, 0)),
    ] + [blk] * len(residuals)
    return pl.pallas_call(
        body,
        grid=grid,
        in_specs=in_specs,
        out_specs=(blk, blk),
        out_shape=(jax.ShapeDtypeStruct((D, NPAD), jnp.float32),
                   jax.ShapeDtypeStruct((D, NPAD), jnp.float32)),
    )(acc, deg, degn, bt, wt, *residuals)


def _final_call(acc, deg, bt):
    grid = (NPAD // BR,)
    blk = pl.BlockSpec((D, BR), lambda i: (0, i))
    return pl.pallas_call(
        _final_body,
        grid=grid,
        in_specs=[
            blk,
            pl.BlockSpec((NW, BR), lambda i: (0, i)),
            pl.BlockSpec((D, 8), lambda i: (0, 0)),
        ],
        out_specs=blk,
        out_shape=jax.ShapeDtypeStruct((D, NPAD), jnp.float32),
    )(acc, deg, bt)


def kernel(x, rest_ast_edge_index, edges_remove_0, edges_remove_1, W, b):
    xt = jnp.pad(x.T, ((0, 0), (0, NPAD - N)))
    wt = W.T
    bt = jnp.broadcast_to(b[:, None], (D, 8))
    src0, dst0, ep0 = _pad_edges(edges_remove_0)
    src1, dst1, ep1 = _pad_edges(edges_remove_1)
    src2, dst2, ep2 = _pad_edges(rest_ast_edge_index)

    deg0, deg1, deg2 = _deg_call(dst0, dst1, dst2, ep0, ep1, ep2)

    g0 = _pre_call(xt, wt, deg0)
    acc0 = _scatter_call(g0, src0, dst0, ep0)
    x1, g1 = _postpre_call(_postpre_body, acc0, deg0, deg1, bt, wt, xt)

    acc1 = _scatter_call(g1, src1, dst1, ep1)
    x2, g2 = _postpre_call(_postpre2_body, acc1, deg1, deg2, bt, wt, x1, xt)

    acc2 = _scatter_call(g2, src2, dst2, ep2)
    out = _final_call(acc2, deg2, bt)
    return out[:, :N].T


# speedup vs baseline: 6.2408x; 1.0005x over previous
import functools
import math

import jax
import jax.numpy as jnp
import numpy as np
from jax import lax
from jax.experimental import pallas as pl
from jax.experimental.pallas import tpu as pltpu
from jax.experimental.pallas import tpu_sc as plsc

N = 10000
D = 256
NC = 2
NS = 16
NW = NC * NS
CH = 512
NPAD = 10240
BR = 1024
FPT = 4
PASSES = D // (NW * FPT)

_SC_PARAMS = pltpu.CompilerParams(needs_layout_passes=False)


def _pad_edges(e):
    ecount = e.shape[1]
    ep = ((ecount + NW * CH - 1) // (NW * CH)) * (NW * CH)
    pad = ep - ecount
    src = jnp.concatenate([e[0], jnp.zeros((pad,), jnp.int32)])
    dst = jnp.concatenate([e[1], jnp.full((pad,), N, jnp.int32)])
    return src, dst, ep


def _deg_hist(w, ep, dst_hbm, out_ref, dstb, hist):
    zeros16 = jnp.zeros((16,), jnp.float32)
    ones16 = jnp.full((16,), 1.0, jnp.float32)

    def zero(k, carry):
        hist[pl.ds(k * 16, 16)] = zeros16
        return carry

    lax.fori_loop(0, NPAD // 16, zero, 0)
    pt = ep // NW

    def chunk(k, carry):
        off = w * pt + k * CH
        pltpu.sync_copy(dst_hbm.at[pl.ds(off, CH)], dstb)
        for g in range(CH // 16):
            dv = dstb[pl.ds(g * 16, 16)]
            plsc.addupdate_scatter(hist, [dv], ones16)
        return carry

    lax.fori_loop(0, pt // CH, chunk, 0)
    pltpu.sync_copy(hist, out_ref)


def _deg_call(dst0, dst1, dst2, ep0, ep1, ep2):
    mesh = plsc.VectorSubcoreMesh(core_axis_name="c", subcore_axis_name="s",
                                  num_cores=NC, num_subcores=NS)
    eps = (ep0, ep1, ep2)

    def body(d0, d1, d2, out0, dstb, hist):
        c = lax.axis_index("c")
        s = lax.axis_index("s")
        w = c * NS + s
        for e, dh in enumerate((d0, d1, d2)):
            _deg_hist(w, eps[e], dh, out0.at[e, w], dstb, hist)

    k = pl.kernel(
        body,
        compiler_params=_SC_PARAMS,
        out_type=jax.ShapeDtypeStruct((3, NW, NPAD), jnp.float32),
        mesh=mesh,
        scratch_types=[
            pltpu.VMEM((CH,), jnp.int32),
            pltpu.VMEM((NPAD,), jnp.float32),
        ],
    )
    degs = k(dst0, dst1, dst2)
    return degs[0], degs[1], degs[2]


SCH = 1024


def _scatter_body(ep, gt_hbm, src_hbm, dst_hbm, out_hbm,
                  srcb, dstb, gts, accs, sems):
    c = lax.axis_index("c")
    s = lax.axis_index("s")
    w = c * NS + s
    nchunks = ep // SCH
    shift = (w * nchunks) // NW

    def off(kk):
        kk = kk + shift
        kk = jnp.where(kk >= nchunks, kk - nchunks, kk)
        return kk * SCH

    def start(b, k):
        pltpu.async_copy(src_hbm.at[pl.ds(off(k), SCH)], srcb.at[b], sems[b])
        pltpu.async_copy(dst_hbm.at[pl.ds(off(k), SCH)], dstb.at[b], sems[b])

    def drain(b):
        pltpu.make_async_copy(src_hbm.at[pl.ds(0, SCH)], srcb.at[b], sems[b]).wait()
        pltpu.make_async_copy(dst_hbm.at[pl.ds(0, SCH)], dstb.at[b], sems[b]).wait()

    for p in range(PASSES):
        fbase = p * (D // PASSES) + w * FPT
        for j in range(FPT):
            pltpu.sync_copy(gt_hbm.at[fbase + j], gts[j])
            pltpu.sync_copy(gt_hbm.at[fbase + j], accs[j])
        for b in range(2):
            start(b, b)

        def pair(k2, carry):
            k = k2 * 2
            for b in range(2):
                drain(b)
                for g in range(SCH // 16):
                    sl = pl.ds(g * 16, 16)
                    sv = srcb[b, sl]
                    dv = dstb[b, sl]
                    for j in range(FPT):
                        vals = plsc.load_gather(gts[j], [sv])
                        plsc.addupdate_scatter(accs[j], [dv], vals)
                nxt = k + b + 2

                @pl.when(nxt < nchunks)
                def _():
                    start(b, nxt)
            return carry

        lax.fori_loop(0, nchunks // 2, pair, 0)
        for j in range(FPT):
            pltpu.sync_copy(accs[j], out_hbm.at[fbase + j])


def _scatter_call(gt, src, dst, ep):
    mesh = plsc.VectorSubcoreMesh(core_axis_name="c", subcore_axis_name="s",
                                  num_cores=NC, num_subcores=NS)

    def body(gt_h, src_h, dst_h, out_h, *scr):
        srcb, dstb = scr[0], scr[1]
        sems = scr[2:4]
        gts = scr[4:4 + FPT]
        accs = scr[4 + FPT:4 + 2 * FPT]
        _scatter_body(ep, gt_h, src_h, dst_h, out_h, srcb, dstb,
                      gts, accs, sems)

    k = pl.kernel(
        body,
        compiler_params=_SC_PARAMS,
        out_type=jax.ShapeDtypeStruct((D, NPAD), jnp.float32),
        mesh=mesh,
        scratch_types=[pltpu.VMEM((2, SCH), jnp.int32)] * 2
        + [pltpu.SemaphoreType.DMA] * 2
        + [pltpu.VMEM((NPAD,), jnp.float32)] * (2 * FPT),
    )
    return k(gt, src, dst)


def _dinv(deg_blk):
    deg = jnp.sum(deg_blk, axis=0) + 1.0
    return lax.rsqrt(jnp.maximum(deg, 1e-12))[None, :]


def _pre_body(x_ref, w_ref, deg_ref, g_ref):
    h = jnp.dot(w_ref[...], x_ref[...], preferred_element_type=jnp.float32)
    g_ref[...] = h * _dinv(deg_ref[...])


def _pre_call(xt, wt, deg):
    grid = (NPAD // BR,)
    blk = pl.BlockSpec((D, BR), lambda i: (0, i))
    return pl.pallas_call(
        _pre_body,
        grid=grid,
        in_specs=[
            blk,
            pl.BlockSpec((D, D), lambda i: (0, 0)),
            pl.BlockSpec((NW, BR), lambda i: (0, i)),
        ],
        out_specs=blk,
        out_shape=jax.ShapeDtypeStruct((D, NPAD), jnp.float32),
    )(xt, wt, deg)


def _ln_gelu(t):
    mu = jnp.mean(t, axis=0, keepdims=True)
    dlt = t - mu
    var = jnp.mean(dlt * dlt, axis=0, keepdims=True)
    y = dlt * lax.rsqrt(var + 1e-5)
    return 0.5 * y * (1.0 + lax.erf(y * np.float32(1.0 / math.sqrt(2.0))))


def _postpre_body(acc_ref, deg_ref, degn_ref, b_ref, w_ref, r1_ref,
                  x_ref, g_ref):
    t = acc_ref[...] * _dinv(deg_ref[...]) + b_ref[:, 0:1]
    xn = _ln_gelu(t) + r1_ref[...]
    x_ref[...] = xn
    h = jnp.dot(w_ref[...], xn, preferred_element_type=jnp.float32)
    g_ref[...] = h * _dinv(degn_ref[...])


def _postpre2_body(acc_ref, deg_ref, degn_ref, b_ref, w_ref, r1_ref, r2_ref,
                   x_ref, g_ref):
    t = acc_ref[...] * _dinv(deg_ref[...]) + b_ref[:, 0:1]
    xn = _ln_gelu(t) + r1_ref[...] + r2_ref[...]
    x_ref[...] = xn
    h = jnp.dot(w_ref[...], xn, preferred_element_type=jnp.float32)
    g_ref[...] = h * _dinv(degn_ref[...])


def _final_body(acc_ref, deg_ref, b_ref, out_ref):
    t = acc_ref[...] * _dinv(deg_ref[...]) + b_ref[:, 0:1]
    out_ref[...] = jnp.maximum(t, 0.0)


def _postpre_call(body, acc, deg, degn, bt, wt, *residuals):
    grid = (NPAD // BR,)
    blk = pl.BlockSpec((D, BR), lambda i: (0, i))
    dblk = pl.BlockSpec((NW, BR), lambda i: (0, i))
    in_specs = [
        blk,
        dblk,
        dblk,
        pl.BlockSpec((D, 8), lambda i: (0, 0)),
        pl.BlockSpec((D, D), lambda i: (0, 0)),
    ] + [blk] * len(residuals)
    return pl.pallas_call(
        body,
        grid=grid,
        in_specs=in_specs,
        out_specs=(blk, blk),
        out_shape=(jax.ShapeDtypeStruct((D, NPAD), jnp.float32),
                   jax.ShapeDtypeStruct((D, NPAD), jnp.float32)),
    )(acc, deg, degn, bt, wt, *residuals)


def _final_call(acc, deg, bt):
    grid = (NPAD // BR,)
    blk = pl.BlockSpec((D, BR), lambda i: (0, i))
    return pl.pallas_call(
        _final_body,
        grid=grid,
        in_specs=[
            blk,
            pl.BlockSpec((NW, BR), lambda i: (0, i)),
            pl.BlockSpec((D, 8), lambda i: (0, 0)),
        ],
        out_specs=blk,
        out_shape=jax.ShapeDtypeStruct((D, NPAD), jnp.float32),
    )(acc, deg, bt)


def kernel(x, rest_ast_edge_index, edges_remove_0, edges_remove_1, W, b):
    xt = jnp.pad(x.T, ((0, 0), (0, NPAD - N)))
    wt = W.T
    bt = jnp.broadcast_to(b[:, None], (D, 8))
    src0, dst0, ep0 = _pad_edges(edges_remove_0)
    src1, dst1, ep1 = _pad_edges(edges_remove_1)
    src2, dst2, ep2 = _pad_edges(rest_ast_edge_index)

    deg0, deg1, deg2 = _deg_call(dst0, dst1, dst2, ep0, ep1, ep2)

    g0 = _pre_call(xt, wt, deg0)
    acc0 = _scatter_call(g0, src0, dst0, ep0)
    x1, g1 = _postpre_call(_postpre_body, acc0, deg0, deg1, bt, wt, xt)

    acc1 = _scatter_call(g1, src1, dst1, ep1)
    x2, g2 = _postpre_call(_postpre2_body, acc1, deg1, deg2, bt, wt, x1, xt)

    acc2 = _scatter_call(g2, src2, dst2, ep2)
    out = _final_call(acc2, deg2, bt)
    return out[:, :N].T


# speedup vs baseline: 9.3311x; 1.4952x over previous
import functools
import math

import jax
import jax.numpy as jnp
import numpy as np
from jax import lax
from jax.experimental import pallas as pl
from jax.experimental.pallas import tpu as pltpu
from jax.experimental.pallas import tpu_sc as plsc

N = 10000
D = 256
NC = 2
NS = 16
NW = NC * NS
CH = 512
NPAD = 10240
BR = 1024
FPT = 4
PASSES = D // (NW * FPT)

_SC_PARAMS = pltpu.CompilerParams(needs_layout_passes=False)


def _pad_edges(e):
    ecount = e.shape[1]
    ep = ((ecount + NW * CH - 1) // (NW * CH)) * (NW * CH)
    pad = ep - ecount
    src = jnp.concatenate([e[0], jnp.zeros((pad,), jnp.int32)])
    dst = jnp.concatenate([e[1], jnp.full((pad,), N, jnp.int32)])
    return src, dst, ep


def _deg_hist(w, ep, dst_hbm, out_ref, dstb, hist):
    zeros16 = jnp.zeros((16,), jnp.float32)
    ones16 = jnp.full((16,), 1.0, jnp.float32)

    def zero(k, carry):
        hist[pl.ds(k * 16, 16)] = zeros16
        return carry

    lax.fori_loop(0, NPAD // 16, zero, 0)
    pt = ep // NW

    def chunk(k, carry):
        off = w * pt + k * CH
        pltpu.sync_copy(dst_hbm.at[pl.ds(off, CH)], dstb)
        for g in range(CH // 16):
            dv = dstb[pl.ds(g * 16, 16)]
            plsc.addupdate_scatter(hist, [dv], ones16)
        return carry

    lax.fori_loop(0, pt // CH, chunk, 0)
    pltpu.sync_copy(hist, out_ref)


def _deg_call(dst0, dst1, dst2, ep0, ep1, ep2):
    mesh = plsc.VectorSubcoreMesh(core_axis_name="c", subcore_axis_name="s",
                                  num_cores=NC, num_subcores=NS)
    eps = (ep0, ep1, ep2)

    def body(d0, d1, d2, out0, dstb, hist):
        c = lax.axis_index("c")
        s = lax.axis_index("s")
        w = c * NS + s
        for e, dh in enumerate((d0, d1, d2)):
            _deg_hist(w, eps[e], dh, out0.at[e, w], dstb, hist)

    k = pl.kernel(
        body,
        compiler_params=_SC_PARAMS,
        out_type=jax.ShapeDtypeStruct((3, NW, NPAD), jnp.float32),
        mesh=mesh,
        scratch_types=[
            pltpu.VMEM((CH,), jnp.int32),
            pltpu.VMEM((NPAD,), jnp.float32),
        ],
    )
    degs = k(dst0, dst1, dst2)
    return degs[0], degs[1], degs[2]


SCH = 1024


def _scatter_body(ep, gt_hbm, src_hbm, dst_hbm, out_hbm,
                  srcb, dstb, gts, accs, sems, ssem):
    c = lax.axis_index("c")
    s = lax.axis_index("s")
    w = c * NS + s
    nchunks = ep // SCH
    shift = (w * nchunks) // NW

    def off(kk):
        kk = kk + shift
        kk = jnp.where(kk >= nchunks, kk - nchunks, kk)
        return kk * SCH

    def start(b, k):
        pltpu.async_copy(src_hbm.at[pl.ds(off(k), SCH)], srcb.at[b], sems[b])
        pltpu.async_copy(dst_hbm.at[pl.ds(off(k), SCH)], dstb.at[b], sems[b])

    def drain(b):
        pltpu.make_async_copy(src_hbm.at[pl.ds(0, SCH)], srcb.at[b], sems[b]).wait()
        pltpu.make_async_copy(dst_hbm.at[pl.ds(0, SCH)], dstb.at[b], sems[b]).wait()

    for p in range(PASSES):
        fbase = p * (D // PASSES) + w * FPT
        for j in range(FPT):
            pltpu.async_copy(gt_hbm.at[fbase + j], gts[j], ssem)
            pltpu.async_copy(gt_hbm.at[fbase + j], accs[j], ssem)
        for b in range(2):
            start(b, b)
        for j in range(FPT):
            pltpu.make_async_copy(gt_hbm.at[fbase + j], gts[j], ssem).wait()
            pltpu.make_async_copy(gt_hbm.at[fbase + j], accs[j], ssem).wait()

        def pair(k2, carry):
            k = k2 * 2
            for b in range(2):
                drain(b)
                for g in range(SCH // 16):
                    sl = pl.ds(g * 16, 16)
                    sv = srcb[b, sl]
                    dv = dstb[b, sl]
                    vals = [plsc.load_gather(gts[j], [sv]) for j in range(FPT)]
                    for j in range(FPT):
                        plsc.addupdate_scatter(accs[j], [dv], vals[j])
                nxt = k + b + 2

                @pl.when(nxt < nchunks)
                def _():
                    start(b, nxt)
            return carry

        lax.fori_loop(0, nchunks // 2, pair, 0)
        for j in range(FPT):
            pltpu.async_copy(accs[j], out_hbm.at[fbase + j], ssem)
        for j in range(FPT):
            pltpu.make_async_copy(accs[j], out_hbm.at[fbase + j], ssem).wait()


def _scatter_call(gt, src, dst, ep):
    mesh = plsc.VectorSubcoreMesh(core_axis_name="c", subcore_axis_name="s",
                                  num_cores=NC, num_subcores=NS)

    def body(gt_h, src_h, dst_h, out_h, *scr):
        srcb, dstb = scr[0], scr[1]
        sems = scr[2:4]
        ssem = scr[4]
        gts = scr[5:5 + FPT]
        accs = scr[5 + FPT:5 + 2 * FPT]
        _scatter_body(ep, gt_h, src_h, dst_h, out_h, srcb, dstb,
                      gts, accs, sems, ssem)

    k = pl.kernel(
        body,
        compiler_params=_SC_PARAMS,
        out_type=jax.ShapeDtypeStruct((D, NPAD), jnp.float32),
        mesh=mesh,
        scratch_types=[pltpu.VMEM((2, SCH), jnp.int32)] * 2
        + [pltpu.SemaphoreType.DMA] * 3
        + [pltpu.VMEM((NPAD,), jnp.float32)] * (2 * FPT),
    )
    return k(gt, src, dst)


def _dinv(deg_blk):
    deg = jnp.sum(deg_blk, axis=0) + 1.0
    return lax.rsqrt(jnp.maximum(deg, 1e-12))[None, :]


def _pre_body(x_ref, w_ref, deg_ref, g_ref):
    h = jnp.dot(w_ref[...], x_ref[...], preferred_element_type=jnp.float32)
    g_ref[...] = h * _dinv(deg_ref[...])


def _pre_call(xt, wt, deg):
    grid = (NPAD // BR,)
    blk = pl.BlockSpec((D, BR), lambda i: (0, i))
    return pl.pallas_call(
        _pre_body,
        grid=grid,
        in_specs=[
            blk,
            pl.BlockSpec((D, D), lambda i: (0, 0)),
            pl.BlockSpec((NW, BR), lambda i: (0, i)),
        ],
        out_specs=blk,
        out_shape=jax.ShapeDtypeStruct((D, NPAD), jnp.float32),
    )(xt, wt, deg)


def _ln_gelu(t):
    mu = jnp.mean(t, axis=0, keepdims=True)
    dlt = t - mu
    var = jnp.mean(dlt * dlt, axis=0, keepdims=True)
    y = dlt * lax.rsqrt(var + 1e-5)
    return 0.5 * y * (1.0 + lax.erf(y * np.float32(1.0 / math.sqrt(2.0))))


def _postpre_body(acc_ref, deg_ref, degn_ref, b_ref, w_ref, r1_ref,
                  x_ref, g_ref):
    t = acc_ref[...] * _dinv(deg_ref[...]) + b_ref[:, 0:1]
    xn = _ln_gelu(t) + r1_ref[...]
    x_ref[...] = xn
    h = jnp.dot(w_ref[...], xn, preferred_element_type=jnp.float32)
    g_ref[...] = h * _dinv(degn_ref[...])


def _postpre2_body(acc_ref, deg_ref, degn_ref, b_ref, w_ref, r1_ref, r2_ref,
                   x_ref, g_ref):
    t = acc_ref[...] * _dinv(deg_ref[...]) + b_ref[:, 0:1]
    xn = _ln_gelu(t) + r1_ref[...] + r2_ref[...]
    x_ref[...] = xn
    h = jnp.dot(w_ref[...], xn, preferred_element_type=jnp.float32)
    g_ref[...] = h * _dinv(degn_ref[...])


def _final_body(acc_ref, deg_ref, b_ref, out_ref):
    t = acc_ref[...] * _dinv(deg_ref[...]) + b_ref[:, 0:1]
    out_ref[...] = jnp.maximum(t, 0.0)


def _postpre_call(body, acc, deg, degn, bt, wt, *residuals):
    grid = (NPAD // BR,)
    blk = pl.BlockSpec((D, BR), lambda i: (0, i))
    dblk = pl.BlockSpec((NW, BR), lambda i: (0, i))
    in_specs = [
        blk,
        dblk,
        dblk,
        pl.BlockSpec((D, 8), lambda i: (0, 0)),
        pl.BlockSpec((D, D), lambda i: (0, 0)),
    ] + [blk] * len(residuals)
    return pl.pallas_call(
        body,
        grid=grid,
        in_specs=in_specs,
        out_specs=(blk, blk),
        out_shape=(jax.ShapeDtypeStruct((D, NPAD), jnp.float32),
                   jax.ShapeDtypeStruct((D, NPAD), jnp.float32)),
    )(acc, deg, degn, bt, wt, *residuals)


def _final_call(acc, deg, bt):
    grid = (NPAD // BR,)
    blk = pl.BlockSpec((D, BR), lambda i: (0, i))
    return pl.pallas_call(
        _final_body,
        grid=grid,
        in_specs=[
            blk,
            pl.BlockSpec((NW, BR), lambda i: (0, i)),
            pl.BlockSpec((D, 8), lambda i: (0, 0)),
        ],
        out_specs=blk,
        out_shape=jax.ShapeDtypeStruct((D, NPAD), jnp.float32),
    )(acc, deg, bt)


def kernel(x, rest_ast_edge_index, edges_remove_0, edges_remove_1, W, b):
    xt = jnp.pad(x.T, ((0, 0), (0, NPAD - N)))
    wt = W.T
    bt = jnp.broadcast_to(b[:, None], (D, 8))
    src0, dst0, ep0 = _pad_edges(edges_remove_0)
    src1, dst1, ep1 = _pad_edges(edges_remove_1)
    src2, dst2, ep2 = _pad_edges(rest_ast_edge_index)

    deg0, deg1, deg2 = _deg_call(dst0, dst1, dst2, ep0, ep1, ep2)

    g0 = _pre_call(xt, wt, deg0)
    acc0 = _scatter_call(g0, src0, dst0, ep0)
    x1, g1 = _postpre_call(_postpre_body, acc0, deg0, deg1, bt, wt, xt)

    acc1 = _scatter_call(g1, src1, dst1, ep1)
    x2, g2 = _postpre_call(_postpre2_body, acc1, deg1, deg2, bt, wt, x1, xt)

    acc2 = _scatter_call(g2, src2, dst2, ep2)
    out = _final_call(acc2, deg2, bt)
    return out[:, :N].T
